# Initial kernel scaffold; baseline (speedup 1.0000x reference)
import jax
import jax.numpy as jnp
from jax.experimental import pallas as pl


def kernel(input_ids, visual_embeds, token_table, type_table, class_embedding, pos_table, ln_gamma, ln_beta):
    raise NotImplementedError("write your pallas kernel here")


# speedup vs baseline: 1.5280x; 1.5280x over previous
import functools

import jax
import jax.numpy as jnp
from jax import lax
from jax.experimental import pallas as pl
from jax.experimental.pallas import tpu as pltpu
from jax.experimental.pallas import tpu_sc as plsc

B = 1024
LT = 50
LV = 49
D = 512
S = LV + 1 + LT

NC = 2
NS = 16
NW = NC * NS
NTOK = B * LT
PER_W = NTOK // NW
CH = 64
NCHUNK = PER_W // CH


def _sc_gather_body(ids_hbm, table_hbm, out_hbm, idx_v, rows_v, sem):
    wid = lax.axis_index("s") * NC + lax.axis_index("c")
    base = wid * PER_W

    def chunk(i, carry):
        off = pl.multiple_of(base + i * CH, CH)
        pltpu.sync_copy(ids_hbm.at[pl.ds(off, CH)], idx_v)
        pltpu.async_copy(table_hbm.at[idx_v], rows_v, sem).wait()
        pltpu.sync_copy(rows_v, out_hbm.at[pl.ds(off, CH)])
        return carry

    lax.fori_loop(0, NCHUNK, chunk, 0)


@jax.jit
def _sc_gather(ids_flat, table):
    mesh = plsc.VectorSubcoreMesh(core_axis_name="c", subcore_axis_name="s")
    fn = functools.partial(
        pl.kernel,
        mesh=mesh,
        out_type=jax.ShapeDtypeStruct((NTOK, D), jnp.float32),
        scratch_types=[
            pltpu.VMEM((CH,), jnp.int32),
            pltpu.VMEM((CH, D), jnp.float32),
            pltpu.SemaphoreType.DMA,
        ],
    )(_sc_gather_body)
    return fn(ids_flat, table)


BB = 16


def _ln(x, gamma, beta):
    mu = jnp.mean(x, axis=-1, keepdims=True)
    xc = x - mu
    var = jnp.mean(xc * xc, axis=-1, keepdims=True)
    return xc * lax.rsqrt(var + 1e-5) * gamma + beta


def _tc_body(g_ref, vis_ref, pos_ref, type_ref, cls_ref, gamma_ref,
             beta_ref, out_ref):
    gamma = gamma_ref[...][None]
    beta = beta_ref[...][None]
    row0 = cls_ref[...] + type_ref[0:1] + pos_ref[0:1]
    y0 = _ln(row0[None], gamma, beta)
    out_ref[:, 0:1, :] = jnp.broadcast_to(y0, (BB, 1, D))
    bvis = (pos_ref[1:LV + 1] + type_ref[0:1])[None]
    out_ref[:, 1:LV + 1, :] = _ln(vis_ref[...] + bvis, gamma, beta)
    btxt = (pos_ref[LV + 1:S] + type_ref[1:2])[None]
    out_ref[:, LV + 1:S, :] = _ln(g_ref[...] + btxt, gamma, beta)


@jax.jit
def _tc_assemble(g3, visual_embeds, pos100, type_table, cls2d, gamma2d,
                 beta2d):
    return pl.pallas_call(
        _tc_body,
        grid=(B // BB,),
        in_specs=[
            pl.BlockSpec((BB, LT, D), lambda i: (i, 0, 0)),
            pl.BlockSpec((BB, LV, D), lambda i: (i, 0, 0)),
            pl.BlockSpec((S, D), lambda i: (0, 0)),
            pl.BlockSpec((2, D), lambda i: (0, 0)),
            pl.BlockSpec((1, D), lambda i: (0, 0)),
            pl.BlockSpec((1, D), lambda i: (0, 0)),
            pl.BlockSpec((1, D), lambda i: (0, 0)),
        ],
        out_specs=pl.BlockSpec((BB, S, D), lambda i: (i, 0, 0)),
        out_shape=jax.ShapeDtypeStruct((B, S, D), jnp.float32),
    )(g3, visual_embeds, pos100, type_table, cls2d, gamma2d, beta2d)


def kernel(input_ids, visual_embeds, token_table, type_table,
           class_embedding, pos_table, ln_gamma, ln_beta):
    ids_flat = input_ids.reshape(-1).astype(jnp.int32)
    g = _sc_gather(ids_flat, token_table)
    g3 = g.reshape(B, LT, D)
    pos100 = pos_table[:S]
    return _tc_assemble(g3, visual_embeds, pos100, type_table,
                        class_embedding[None], ln_gamma[None], ln_beta[None])
